# baseline (speedup 1.0000x reference)
import functools

import jax
import jax.numpy as jnp
from jax import lax
from jax.experimental import pallas as pl
from jax.experimental.pallas import tpu as pltpu
from jax.experimental.pallas import tpu_sc as plsc

EPS = 1e-8
KB = 6144


def _topk_body(x_ref, t_ref, v_ref, g_ref,
               d1_ref, i0_ref, thr_ref,
               qn_ref, s_ref, rv0, ri0, rv1, *, K, NB):
    k = pl.program_id(0)
    B = x_ref.shape[0]

    @pl.when(k == 0)
    def _init():
        x = x_ref[...]
        xn = jnp.sqrt(jnp.sum(x * x, axis=1, keepdims=True))
        qn_ref[...] = x / jnp.maximum(xn, EPS)
        t = t_ref[...]
        num = jnp.sum(x * t, axis=1, keepdims=True)
        tn = jnp.sqrt(jnp.sum(t * t, axis=1, keepdims=True))
        ts = num / jnp.maximum(xn * tn, EPS)
        thr_ref[...] = g_ref[0] + jnp.sqrt(jnp.maximum(2.0 * (1.0 - ts), 0.0))
        rv0[...] = jnp.full((B, 1), -jnp.inf, jnp.float32)
        rv1[...] = jnp.full((B, 1), -jnp.inf, jnp.float32)
        ri0[...] = jnp.zeros((B, 1), jnp.int32)

    vb = v_ref[...]
    vnrm = jnp.sqrt(jnp.sum(vb * vb, axis=1, keepdims=True))
    vn = vb / jnp.maximum(vnrm, EPS)
    s_ref[...] = lax.dot_general(
        qn_ref[...], vn, (((1,), (1,)), ((), ())),
        preferred_element_type=jnp.float32)

    col = lax.broadcasted_iota(jnp.int32, (B, KB), 1).astype(jnp.float32)
    @pl.when(k == NB - 1)
    def _mask_tail():
        lim = float(K - (NB - 1) * KB)
        s_ref[...] = jnp.where(col < lim, s_ref[...], -jnp.inf)

    s = s_ref[...]
    BIGF = jnp.float32(2.0 * KB)
    m1 = jnp.max(s, axis=1, keepdims=True)
    i1 = jnp.min(jnp.where(s == m1, col, BIGF), axis=1, keepdims=True)
    m2 = jnp.max(jnp.where(col == i1, -jnp.inf, s), axis=1, keepdims=True)

    a = m1 > rv0[...]
    rv1[...] = jnp.maximum(jnp.where(a, rv0[...], rv1[...]),
                           jnp.where(a, m2, m1))
    ri0[...] = jnp.where(a, i1.astype(jnp.int32) + k * KB, ri0[...])
    rv0[...] = jnp.where(a, m1, rv0[...])

    @pl.when(k == NB - 1)
    def _fin():
        d1_ref[...] = jnp.sqrt(jnp.maximum(2.0 * (1.0 - rv1[...]), 0.0))
        i0_ref[...] = ri0[...]


def _topk_call(x, t, v, g):
    B, D = x.shape
    K = v.shape[0]
    NB = pl.cdiv(K, KB)
    out = pl.pallas_call(
        functools.partial(_topk_body, K=K, NB=NB),
        grid=(NB,),
        in_specs=[
            pl.BlockSpec((B, D), lambda k: (0, 0)),
            pl.BlockSpec((B, D), lambda k: (0, 0)),
            pl.BlockSpec((KB, D), lambda k: (k, 0)),
            pl.BlockSpec(memory_space=pltpu.SMEM),
        ],
        out_specs=[
            pl.BlockSpec((B, 1), lambda k: (0, 0)),
            pl.BlockSpec((B, 1), lambda k: (0, 0)),
            pl.BlockSpec((B, 1), lambda k: (0, 0)),
        ],
        out_shape=[
            jax.ShapeDtypeStruct((B, 1), jnp.float32),
            jax.ShapeDtypeStruct((B, 1), jnp.int32),
            jax.ShapeDtypeStruct((B, 1), jnp.float32),
        ],
        scratch_shapes=[
            pltpu.VMEM((B, D), jnp.float32),
            pltpu.VMEM((B, KB), jnp.float32),
            pltpu.VMEM((B, 1), jnp.float32),
            pltpu.VMEM((B, 1), jnp.int32),
            pltpu.VMEM((B, 1), jnp.float32),
        ],
        compiler_params=pltpu.CompilerParams(
            dimension_semantics=("arbitrary",)),
    )(x, t, v, g.reshape(1))
    return out


def _sc_gather(table, idx):
    V, D = table.shape
    B = idx.shape[0]
    info = plsc.get_sparse_core_info()
    NW = info.num_cores * info.num_subcores
    b_per_w = B // NW
    mesh = plsc.VectorSubcoreMesh(core_axis_name="c", subcore_axis_name="s")

    @functools.partial(
        pl.kernel, mesh=mesh,
        out_type=jax.ShapeDtypeStruct((B, D), jnp.float32),
        scratch_types=[
            pltpu.VMEM((b_per_w,), jnp.int32),
            pltpu.VMEM((b_per_w, D), jnp.float32),
            pltpu.SemaphoreType.DMA,
        ],
    )
    def k(table_hbm, idx_hbm, out_hbm, idx_v, rows_v, sem):
        wid = lax.axis_index("s") * info.num_cores + lax.axis_index("c")
        base = wid * b_per_w
        pltpu.sync_copy(idx_hbm.at[pl.ds(base, b_per_w)], idx_v)
        pltpu.async_copy(table_hbm.at[idx_v], rows_v, sem).wait()
        pltpu.sync_copy(rows_v, out_hbm.at[pl.ds(base, b_per_w)])

    return k(table, idx)


def _finish_body(x_ref, g_ref, t_ref, d1_ref, thr_ref, o_ref):
    B = g_ref.shape[0]
    x = x_ref[...]
    v0 = g_ref[...]
    num = jnp.sum(x * v0, axis=1, keepdims=True)
    na = jnp.sqrt(jnp.sum(x * x, axis=1, keepdims=True))
    nb = jnp.sqrt(jnp.sum(v0 * v0, axis=1, keepdims=True))
    sim = num / jnp.maximum(na * nb, EPS)
    d0 = jnp.sqrt(jnp.maximum(2.0 * (1.0 - sim), 0.0))
    eq = jnp.all(v0 == t_ref[...], axis=1, keepdims=True)
    negd = jnp.where(eq, d1_ref[...], d0)
    cost = jnp.maximum(thr_ref[...] - negd, 0.0) * 2.0
    o_ref[...] = (jnp.sum(cost) / B).reshape(1, 1)


def _finish_call(x, gathered, t, d1, thr):
    out = pl.pallas_call(
        _finish_body,
        out_shape=jax.ShapeDtypeStruct((1, 1), jnp.float32),
    )(x, gathered, t, d1, thr)
    return out[0, 0]


def kernel(input, target, veclist, gamma):
    d1, i0, thr = _topk_call(input, target, veclist,
                             jnp.asarray(gamma, jnp.float32))
    gathered = _sc_gather(veclist, i0.reshape(-1))
    return _finish_call(input, gathered, target, d1, thr)


# speedup vs baseline: 1.0321x; 1.0321x over previous
import functools

import jax
import jax.numpy as jnp
from jax import lax
from jax.experimental import pallas as pl
from jax.experimental.pallas import tpu as pltpu
from jax.experimental.pallas import tpu_sc as plsc

EPS = 1e-8
KB = 5120


def _topk_body(x_ref, t_ref, v_ref, g_ref,
               d1_ref, i0_ref, thr_ref,
               qn_ref, s_ref, rv0, ri0, rv1, *, K, NB):
    k = pl.program_id(0)
    B = x_ref.shape[0]

    @pl.when(k == 0)
    def _init():
        x = x_ref[...]
        xn = jnp.sqrt(jnp.sum(x * x, axis=1, keepdims=True))
        qn_ref[...] = x / jnp.maximum(xn, EPS)
        t = t_ref[...]
        num = jnp.sum(x * t, axis=1, keepdims=True)
        tn = jnp.sqrt(jnp.sum(t * t, axis=1, keepdims=True))
        ts = num / jnp.maximum(xn * tn, EPS)
        thr_ref[...] = g_ref[0] + jnp.sqrt(jnp.maximum(2.0 * (1.0 - ts), 0.0))
        rv0[...] = jnp.full((B, 1), -jnp.inf, jnp.float32)
        rv1[...] = jnp.full((B, 1), -jnp.inf, jnp.float32)
        ri0[...] = jnp.zeros((B, 1), jnp.int32)

    vb = v_ref[...]
    vnrm = jnp.sqrt(jnp.sum(vb * vb, axis=1, keepdims=True))
    vn = vb / jnp.maximum(vnrm, EPS)
    s_ref[...] = lax.dot_general(
        qn_ref[...], vn, (((1,), (1,)), ((), ())),
        preferred_element_type=jnp.float32)

    col = lax.broadcasted_iota(jnp.int32, (B, KB), 1).astype(jnp.float32)
    @pl.when(k == NB - 1)
    def _mask_tail():
        lim = float(K - (NB - 1) * KB)
        s_ref[...] = jnp.where(col < lim, s_ref[...], -jnp.inf)

    s = s_ref[...]
    BIGF = jnp.float32(2.0 * KB)
    m1 = jnp.max(s, axis=1, keepdims=True)
    i1 = jnp.min(jnp.where(s == m1, col, BIGF), axis=1, keepdims=True)
    m2 = jnp.max(jnp.where(col == i1, -jnp.inf, s), axis=1, keepdims=True)

    a = m1 > rv0[...]
    rv1[...] = jnp.maximum(jnp.where(a, rv0[...], rv1[...]),
                           jnp.where(a, m2, m1))
    ri0[...] = jnp.where(a, i1.astype(jnp.int32) + k * KB, ri0[...])
    rv0[...] = jnp.where(a, m1, rv0[...])

    @pl.when(k == NB - 1)
    def _fin():
        d1_ref[...] = jnp.sqrt(jnp.maximum(2.0 * (1.0 - rv1[...]), 0.0))
        i0_ref[...] = ri0[...]


def _topk_call(x, t, v, g):
    B, D = x.shape
    K = v.shape[0]
    NB = pl.cdiv(K, KB)
    out = pl.pallas_call(
        functools.partial(_topk_body, K=K, NB=NB),
        grid=(NB,),
        in_specs=[
            pl.BlockSpec((B, D), lambda k: (0, 0)),
            pl.BlockSpec((B, D), lambda k: (0, 0)),
            pl.BlockSpec((KB, D), lambda k: (k, 0)),
            pl.BlockSpec(memory_space=pltpu.SMEM),
        ],
        out_specs=[
            pl.BlockSpec((B, 1), lambda k: (0, 0)),
            pl.BlockSpec((B, 1), lambda k: (0, 0)),
            pl.BlockSpec((B, 1), lambda k: (0, 0)),
        ],
        out_shape=[
            jax.ShapeDtypeStruct((B, 1), jnp.float32),
            jax.ShapeDtypeStruct((B, 1), jnp.int32),
            jax.ShapeDtypeStruct((B, 1), jnp.float32),
        ],
        scratch_shapes=[
            pltpu.VMEM((B, D), jnp.float32),
            pltpu.VMEM((B, KB), jnp.float32),
            pltpu.VMEM((B, 1), jnp.float32),
            pltpu.VMEM((B, 1), jnp.int32),
            pltpu.VMEM((B, 1), jnp.float32),
        ],
        compiler_params=pltpu.CompilerParams(
            dimension_semantics=("arbitrary",)),
    )(x, t, v, g.reshape(1))
    return out


def _sc_gather(table, idx):
    V, D = table.shape
    B = idx.shape[0]
    info = plsc.get_sparse_core_info()
    NW = info.num_cores * info.num_subcores
    b_per_w = B // NW
    mesh = plsc.VectorSubcoreMesh(core_axis_name="c", subcore_axis_name="s")

    @functools.partial(
        pl.kernel, mesh=mesh,
        out_type=jax.ShapeDtypeStruct((B, D), jnp.float32),
        scratch_types=[
            pltpu.VMEM((b_per_w,), jnp.int32),
            pltpu.VMEM((b_per_w, D), jnp.float32),
            pltpu.SemaphoreType.DMA,
        ],
    )
    def k(table_hbm, idx_hbm, out_hbm, idx_v, rows_v, sem):
        wid = lax.axis_index("s") * info.num_cores + lax.axis_index("c")
        base = wid * b_per_w
        pltpu.sync_copy(idx_hbm.at[pl.ds(base, b_per_w)], idx_v)
        pltpu.async_copy(table_hbm.at[idx_v], rows_v, sem).wait()
        pltpu.sync_copy(rows_v, out_hbm.at[pl.ds(base, b_per_w)])

    return k(table, idx)


def _finish_body(x_ref, g_ref, t_ref, d1_ref, thr_ref, o_ref):
    B = g_ref.shape[0]
    x = x_ref[...]
    v0 = g_ref[...]
    num = jnp.sum(x * v0, axis=1, keepdims=True)
    na = jnp.sqrt(jnp.sum(x * x, axis=1, keepdims=True))
    nb = jnp.sqrt(jnp.sum(v0 * v0, axis=1, keepdims=True))
    sim = num / jnp.maximum(na * nb, EPS)
    d0 = jnp.sqrt(jnp.maximum(2.0 * (1.0 - sim), 0.0))
    eq = jnp.all(v0 == t_ref[...], axis=1, keepdims=True)
    negd = jnp.where(eq, d1_ref[...], d0)
    cost = jnp.maximum(thr_ref[...] - negd, 0.0) * 2.0
    o_ref[...] = (jnp.sum(cost) / B).reshape(1, 1)


def _finish_call(x, gathered, t, d1, thr):
    out = pl.pallas_call(
        _finish_body,
        out_shape=jax.ShapeDtypeStruct((1, 1), jnp.float32),
    )(x, gathered, t, d1, thr)
    return out[0, 0]


def kernel(input, target, veclist, gamma):
    d1, i0, thr = _topk_call(input, target, veclist,
                             jnp.asarray(gamma, jnp.float32))
    gathered = _sc_gather(veclist, i0.reshape(-1))
    return _finish_call(input, gathered, target, d1, thr)
